# baseline (device time: 151607 ns/iter reference)
import jax
import jax.numpy as jnp
from jax import lax
from jax.experimental import pallas as pl
from jax.experimental.pallas import tpu as pltpu

N_DEV = 4


def kernel(A, B):
    m, k = A.shape
    _, n = B.shape

    def body(a_ref, b_ref, out_ref, comm_ref, send_sems, recv_sems):
        my = lax.axis_index("i")
        left = (my + N_DEV - 1) % N_DEV
        right = (my + 1) % N_DEV

        barrier = pltpu.get_barrier_semaphore()
        for nbr in (left, right):
            pl.semaphore_signal(
                barrier, inc=1,
                device_id=(nbr,), device_id_type=pl.DeviceIdType.MESH,
            )
        pl.semaphore_wait(barrier, 2)

        partial = jnp.dot(
            a_ref[...].astype(jnp.bfloat16),
            b_ref[...].astype(jnp.bfloat16),
            preferred_element_type=jnp.float32,
        )
        comm_ref[0] = partial
        acc = partial

        for h in range(N_DEV - 1):
            rdma = pltpu.make_async_remote_copy(
                src_ref=comm_ref.at[h],
                dst_ref=comm_ref.at[h + 1],
                send_sem=send_sems.at[h],
                recv_sem=recv_sems.at[h],
                device_id=(right,),
                device_id_type=pl.DeviceIdType.MESH,
            )
            rdma.start()
            rdma.wait()
            acc = acc + comm_ref[h + 1]

        z = acc
        out_ref[...] = 0.5 * z * (
            1.0 + jnp.tanh(0.7978845608 * (z + 0.044715 * z * z * z))
        )

    return pl.pallas_call(
        body,
        out_shape=jax.ShapeDtypeStruct((m, n), jnp.float32),
        in_specs=[
            pl.BlockSpec(memory_space=pltpu.VMEM),
            pl.BlockSpec(memory_space=pltpu.VMEM),
        ],
        out_specs=pl.BlockSpec(memory_space=pltpu.VMEM),
        scratch_shapes=[
            pltpu.VMEM((N_DEV, m, n), jnp.float32),
            pltpu.SemaphoreType.DMA((N_DEV - 1,)),
            pltpu.SemaphoreType.DMA((N_DEV - 1,)),
        ],
        compiler_params=pltpu.CompilerParams(collective_id=0),
    )(A, B)


# device time: 33785 ns/iter; 4.4874x vs baseline; 4.4874x over previous
import jax
import jax.numpy as jnp
from jax import lax
from jax.experimental import pallas as pl
from jax.experimental.pallas import tpu as pltpu

N_DEV = 4


def _gelu(z):
    return 0.5 * z * (1.0 + jnp.tanh(0.7978845608 * (z + 0.044715 * z * z * z)))


def kernel(A, B):
    m, k = A.shape
    _, n = B.shape
    mh, nh = m // 2, n // 2
    mq = m // 4
    f32, bf16 = jnp.float32, jnp.bfloat16

    def body(
        a_ref, b_ref, out_ref,
        z0_ref, z1_ref,
        s1a_ref, s1b_ref, r1a_ref, r1b_ref,
        u0_ref, u1_ref,
        s2a_ref, s2b_ref, r2a_ref, r2b_ref,
        g0_ref, g1_ref, r3a_ref, r3b_ref,
        w0_ref, w1_ref, r4a_ref, r4b_ref,
        send_sems, recv_sems,
    ):
        my = lax.axis_index("i")
        mx = my // 2
        myy = (my ^ (my >> 1)) & 1
        xn = my ^ 3
        yn = my ^ 1

        rx_me = mh * mx
        rx_nb = mh * (1 - mx)
        ry_me = mh * myy
        ry_nb = mh * (1 - myy)

        barrier = pltpu.get_barrier_semaphore()
        for nbr in (xn, yn):
            pl.semaphore_signal(
                barrier, inc=1,
                device_id=(nbr,), device_id_type=pl.DeviceIdType.MESH,
            )
        pl.semaphore_wait(barrier, 2)

        def exchange(src, dst, ph, path, peer):
            return pltpu.make_async_remote_copy(
                src_ref=src, dst_ref=dst,
                send_sem=send_sems.at[ph, path],
                recv_sem=recv_sems.at[ph, path],
                device_id=(peer,), device_id_type=pl.DeviceIdType.MESH,
            )

        a_bf = a_ref[...].astype(bf16)

        z0_ref[...] = jnp.dot(
            a_bf, b_ref[:, 0:nh].astype(bf16), preferred_element_type=f32
        ).astype(bf16)
        s1a_ref[...] = z0_ref[pl.ds(rx_nb, mh), :]
        rdma1a = exchange(s1a_ref, r1a_ref, 0, 0, xn)
        rdma1a.start()

        z1_ref[...] = jnp.dot(
            a_bf, b_ref[:, nh:n].astype(bf16), preferred_element_type=f32
        ).astype(bf16)
        s1b_ref[...] = z1_ref[pl.ds(ry_nb, mh), :]
        rdma1b = exchange(s1b_ref, r1b_ref, 0, 1, yn)
        rdma1b.start()

        rdma1a.wait()
        u0_ref[...] = (
            z0_ref[pl.ds(rx_me, mh), :].astype(f32) + r1a_ref[...].astype(f32)
        ).astype(bf16)
        s2a_ref[...] = u0_ref[pl.ds(mq * (1 - myy), mq), :]
        rdma2a = exchange(s2a_ref, r2a_ref, 1, 0, yn)
        rdma2a.start()

        rdma1b.wait()
        u1_ref[...] = (
            z1_ref[pl.ds(ry_me, mh), :].astype(f32) + r1b_ref[...].astype(f32)
        ).astype(bf16)
        s2b_ref[...] = u1_ref[pl.ds(mq * (1 - mx), mq), :]
        rdma2b = exchange(s2b_ref, r2b_ref, 1, 1, xn)
        rdma2b.start()

        rdma2a.wait()
        v0 = u0_ref[pl.ds(mq * myy, mq), :].astype(f32) + r2a_ref[...].astype(f32)
        g0_ref[...] = _gelu(v0).astype(bf16)
        rdma3a = exchange(g0_ref, r3a_ref, 2, 0, yn)
        rdma3a.start()

        rdma2b.wait()
        v1 = u1_ref[pl.ds(mq * mx, mq), :].astype(f32) + r2b_ref[...].astype(f32)
        g1_ref[...] = _gelu(v1).astype(bf16)
        rdma3b = exchange(g1_ref, r3b_ref, 2, 1, xn)
        rdma3b.start()

        w0_ref[pl.ds(mq * myy, mq), :] = g0_ref[...]
        w1_ref[pl.ds(mq * mx, mq), :] = g1_ref[...]

        rdma3a.wait()
        w0_ref[pl.ds(mq * (1 - myy), mq), :] = r3a_ref[...]
        rdma4a = exchange(w0_ref, r4a_ref, 3, 0, xn)
        rdma4a.start()

        rdma3b.wait()
        w1_ref[pl.ds(mq * (1 - mx), mq), :] = r3b_ref[...]
        rdma4b = exchange(w1_ref, r4b_ref, 3, 1, yn)
        rdma4b.start()

        out_ref[pl.ds(rx_me, mh), 0:nh] = w0_ref[...].astype(f32)
        out_ref[pl.ds(ry_me, mh), nh:n] = w1_ref[...].astype(f32)

        rdma4a.wait()
        out_ref[pl.ds(rx_nb, mh), 0:nh] = r4a_ref[...].astype(f32)
        rdma4b.wait()
        out_ref[pl.ds(ry_nb, mh), nh:n] = r4b_ref[...].astype(f32)

    return pl.pallas_call(
        body,
        out_shape=jax.ShapeDtypeStruct((m, n), f32),
        in_specs=[
            pl.BlockSpec(memory_space=pltpu.VMEM),
            pl.BlockSpec(memory_space=pltpu.VMEM),
        ],
        out_specs=pl.BlockSpec(memory_space=pltpu.VMEM),
        scratch_shapes=[
            pltpu.VMEM((m, nh), bf16),
            pltpu.VMEM((m, nh), bf16),
            pltpu.VMEM((mh, nh), bf16),
            pltpu.VMEM((mh, nh), bf16),
            pltpu.VMEM((mh, nh), bf16),
            pltpu.VMEM((mh, nh), bf16),
            pltpu.VMEM((mh, nh), bf16),
            pltpu.VMEM((mh, nh), bf16),
            pltpu.VMEM((mq, nh), bf16),
            pltpu.VMEM((mq, nh), bf16),
            pltpu.VMEM((mq, nh), bf16),
            pltpu.VMEM((mq, nh), bf16),
            pltpu.VMEM((mq, nh), bf16),
            pltpu.VMEM((mq, nh), bf16),
            pltpu.VMEM((mq, nh), bf16),
            pltpu.VMEM((mq, nh), bf16),
            pltpu.VMEM((mh, nh), bf16),
            pltpu.VMEM((mh, nh), bf16),
            pltpu.VMEM((mh, nh), bf16),
            pltpu.VMEM((mh, nh), bf16),
            pltpu.SemaphoreType.DMA((4, 2)),
            pltpu.SemaphoreType.DMA((4, 2)),
        ],
        compiler_params=pltpu.CompilerParams(collective_id=0),
    )(A, B)


# device time: 32306 ns/iter; 4.6928x vs baseline; 1.0458x over previous
import jax
import jax.numpy as jnp
from jax import lax
from jax.experimental import pallas as pl
from jax.experimental.pallas import tpu as pltpu

N_DEV = 4


def _gelu(z):
    return 0.5 * z * (1.0 + jnp.tanh(0.7978845608 * (z + 0.044715 * z * z * z)))


def kernel(A, B):
    m, k = A.shape
    _, n = B.shape
    mh, nh = m // 2, n // 2
    mq = m // 4
    f32, bf16 = jnp.float32, jnp.bfloat16

    def body(
        a_ref, b_ref, out_ref,
        s1a_ref, s1b_ref, zk0_ref, zk1_ref, r1a_ref, r1b_ref,
        su0_ref, su1_ref, r2a_ref, r2b_ref,
        g0_ref, g1_ref, r3a_ref, r3b_ref,
        r4a_ref, r4b_ref,
        send_sems, recv_sems,
    ):
        my = lax.axis_index("i")
        mx = my // 2
        myy = (my ^ (my >> 1)) & 1
        xn = my ^ 3
        yn = my ^ 1

        rx_me = mh * mx
        rx_nb = mh * (1 - mx)
        ry_me = mh * myy
        ry_nb = mh * (1 - myy)
        o_s0 = mq * (1 - myy)
        o_k0 = mq * myy
        o_s1 = mq * (1 - mx)
        o_k1 = mq * mx

        barrier = pltpu.get_barrier_semaphore()
        for nbr in (xn, yn):
            pl.semaphore_signal(
                barrier, inc=1,
                device_id=(nbr,), device_id_type=pl.DeviceIdType.MESH,
            )
        pl.semaphore_wait(barrier, 2)

        def exchange(src, dst, ph, path, peer):
            return pltpu.make_async_remote_copy(
                src_ref=src, dst_ref=dst,
                send_sem=send_sems.at[ph, path],
                recv_sem=recv_sems.at[ph, path],
                device_id=(peer,), device_id_type=pl.DeviceIdType.MESH,
            )

        b0 = b_ref[:, 0:nh].astype(bf16)
        b1 = b_ref[:, nh:n].astype(bf16)

        s1a_ref[...] = jnp.dot(
            a_ref[pl.ds(rx_nb, mh), :].astype(bf16), b0,
            preferred_element_type=f32,
        ).astype(bf16)
        rdma1a = exchange(s1a_ref, r1a_ref, 0, 0, xn)
        rdma1a.start()

        s1b_ref[...] = jnp.dot(
            a_ref[pl.ds(ry_nb, mh), :].astype(bf16), b1,
            preferred_element_type=f32,
        ).astype(bf16)
        rdma1b = exchange(s1b_ref, r1b_ref, 0, 1, yn)
        rdma1b.start()

        zk0_ref[...] = jnp.dot(
            a_ref[pl.ds(rx_me, mh), :].astype(bf16), b0,
            preferred_element_type=f32,
        ).astype(bf16)
        zk1_ref[...] = jnp.dot(
            a_ref[pl.ds(ry_me, mh), :].astype(bf16), b1,
            preferred_element_type=f32,
        ).astype(bf16)

        rdma1a.wait()
        su0_ref[...] = (
            zk0_ref[pl.ds(o_s0, mq), :].astype(f32)
            + r1a_ref[pl.ds(o_s0, mq), :].astype(f32)
        ).astype(bf16)
        rdma2a = exchange(su0_ref, r2a_ref, 1, 0, yn)
        rdma2a.start()

        rdma1b.wait()
        su1_ref[...] = (
            zk1_ref[pl.ds(o_s1, mq), :].astype(f32)
            + r1b_ref[pl.ds(o_s1, mq), :].astype(f32)
        ).astype(bf16)
        rdma2b = exchange(su1_ref, r2b_ref, 1, 1, xn)
        rdma2b.start()

        k0 = (
            zk0_ref[pl.ds(o_k0, mq), :].astype(f32)
            + r1a_ref[pl.ds(o_k0, mq), :].astype(f32)
        )
        k1 = (
            zk1_ref[pl.ds(o_k1, mq), :].astype(f32)
            + r1b_ref[pl.ds(o_k1, mq), :].astype(f32)
        )

        rdma2a.wait()
        w0 = _gelu(k0 + r2a_ref[...].astype(f32))
        g0_ref[...] = w0.astype(bf16)
        rdma3a = exchange(g0_ref, r3a_ref, 2, 0, yn)
        rdma3a.start()
        rdma4a1 = exchange(g0_ref, r4a_ref.at[0], 3, 0, xn)
        rdma4a1.start()
        out_ref[pl.ds(rx_me + o_k0, mq), 0:nh] = w0

        rdma2b.wait()
        w1 = _gelu(k1 + r2b_ref[...].astype(f32))
        g1_ref[...] = w1.astype(bf16)
        rdma3b = exchange(g1_ref, r3b_ref, 2, 1, xn)
        rdma3b.start()
        rdma4b1 = exchange(g1_ref, r4b_ref.at[0], 3, 1, yn)
        rdma4b1.start()
        out_ref[pl.ds(ry_me + o_k1, mq), nh:n] = w1

        rdma3a.wait()
        rdma4a2 = exchange(r3a_ref, r4a_ref.at[1], 4, 0, xn)
        rdma4a2.start()
        out_ref[pl.ds(rx_me + o_s0, mq), 0:nh] = r3a_ref[...].astype(f32)

        rdma3b.wait()
        rdma4b2 = exchange(r3b_ref, r4b_ref.at[1], 4, 1, yn)
        rdma4b2.start()
        out_ref[pl.ds(ry_me + o_s1, mq), nh:n] = r3b_ref[...].astype(f32)

        rdma4a1.wait()
        out_ref[pl.ds(rx_nb + o_k0, mq), 0:nh] = r4a_ref[0].astype(f32)
        rdma4b1.wait()
        out_ref[pl.ds(ry_nb + o_k1, mq), nh:n] = r4b_ref[0].astype(f32)
        rdma4a2.wait()
        out_ref[pl.ds(rx_nb + o_s0, mq), 0:nh] = r4a_ref[1].astype(f32)
        rdma4b2.wait()
        out_ref[pl.ds(ry_nb + o_s1, mq), nh:n] = r4b_ref[1].astype(f32)

    return pl.pallas_call(
        body,
        out_shape=jax.ShapeDtypeStruct((m, n), f32),
        in_specs=[
            pl.BlockSpec(memory_space=pltpu.VMEM),
            pl.BlockSpec(memory_space=pltpu.VMEM),
        ],
        out_specs=pl.BlockSpec(memory_space=pltpu.VMEM),
        scratch_shapes=[
            pltpu.VMEM((mh, nh), bf16),
            pltpu.VMEM((mh, nh), bf16),
            pltpu.VMEM((mh, nh), bf16),
            pltpu.VMEM((mh, nh), bf16),
            pltpu.VMEM((mh, nh), bf16),
            pltpu.VMEM((mh, nh), bf16),
            pltpu.VMEM((mq, nh), bf16),
            pltpu.VMEM((mq, nh), bf16),
            pltpu.VMEM((mq, nh), bf16),
            pltpu.VMEM((mq, nh), bf16),
            pltpu.VMEM((mq, nh), bf16),
            pltpu.VMEM((mq, nh), bf16),
            pltpu.VMEM((mq, nh), bf16),
            pltpu.VMEM((mq, nh), bf16),
            pltpu.VMEM((2, mq, nh), bf16),
            pltpu.VMEM((2, mq, nh), bf16),
            pltpu.SemaphoreType.DMA((5, 2)),
            pltpu.SemaphoreType.DMA((5, 2)),
        ],
        compiler_params=pltpu.CompilerParams(collective_id=0),
    )(A, B)


# device time: 28529 ns/iter; 5.3141x vs baseline; 1.1324x over previous
import jax
import jax.numpy as jnp
from jax import lax
from jax.experimental import pallas as pl
from jax.experimental.pallas import tpu as pltpu

N_DEV = 4
G = 2


def _gelu(z):
    return 0.5 * z * (1.0 + jnp.tanh(0.7978845608 * (z + 0.044715 * z * z * z)))


def kernel(A, B):
    m, k = A.shape
    _, n = B.shape
    mh, nh = m // 2, n // 2
    mq = m // 4
    cw = nh // G
    f32, bf16 = jnp.float32, jnp.bfloat16

    def body(
        a_ref, b_ref, out_ref,
        s1a_ref, s1b_ref, zka_ref, zkb_ref, r1a_ref, r1b_ref,
        sua_ref, sub_ref, r2a_ref, r2b_ref,
        ga_ref, gb_ref, r3a_ref, r3b_ref,
        r4a_ref, r4b_ref,
        send_sems, recv_sems,
    ):
        my = lax.axis_index("i")
        mx = my // 2
        myy = (my ^ (my >> 1)) & 1
        xn = my ^ 3
        yn = my ^ 1

        rx_me = mh * mx
        rx_nb = mh * (1 - mx)
        ry_me = mh * myy
        ry_nb = mh * (1 - myy)
        o_sa = mq * (1 - myy)
        o_ka = mq * myy
        o_sb = mq * (1 - mx)
        o_kb = mq * mx
        ca = [slice(g * cw, (g + 1) * cw) for g in range(G)]
        cb = [slice(nh + g * cw, nh + (g + 1) * cw) for g in range(G)]

        barrier = pltpu.get_barrier_semaphore()
        for nbr in (xn, yn):
            pl.semaphore_signal(
                barrier, inc=1,
                device_id=(nbr,), device_id_type=pl.DeviceIdType.MESH,
            )
        pl.semaphore_wait(barrier, 2)

        def exchange(src, dst, ph, path, g, peer):
            return pltpu.make_async_remote_copy(
                src_ref=src, dst_ref=dst,
                send_sem=send_sems.at[ph, path, g],
                recv_sem=recv_sems.at[ph, path, g],
                device_id=(peer,), device_id_type=pl.DeviceIdType.MESH,
            )

        a_xnb = a_ref[pl.ds(rx_nb, mh), :].astype(bf16)
        a_ynb = a_ref[pl.ds(ry_nb, mh), :].astype(bf16)
        a_xme = a_ref[pl.ds(rx_me, mh), :].astype(bf16)
        a_yme = a_ref[pl.ds(ry_me, mh), :].astype(bf16)
        b_bf = b_ref[...].astype(bf16)

        p1 = {}
        for g in range(G):
            s1a_ref[g] = jnp.dot(
                a_xnb, b_bf[:, ca[g]], preferred_element_type=f32
            ).astype(bf16)
            p1["a", g] = exchange(
                s1a_ref.at[g], r1a_ref.at[g], 0, 0, g, xn
            )
            p1["a", g].start()
            s1b_ref[g] = jnp.dot(
                a_ynb, b_bf[:, cb[g]], preferred_element_type=f32
            ).astype(bf16)
            p1["b", g] = exchange(
                s1b_ref.at[g], r1b_ref.at[g], 0, 1, g, yn
            )
            p1["b", g].start()
        for g in range(G):
            zka_ref[g] = jnp.dot(
                a_xme, b_bf[:, ca[g]], preferred_element_type=f32
            ).astype(bf16)
            zkb_ref[g] = jnp.dot(
                a_yme, b_bf[:, cb[g]], preferred_element_type=f32
            ).astype(bf16)

        p2 = {}
        for g in range(G):
            p1["a", g].wait()
            sua_ref[g] = (
                zka_ref[g, pl.ds(o_sa, mq), :].astype(f32)
                + r1a_ref[g, pl.ds(o_sa, mq), :].astype(f32)
            ).astype(bf16)
            p2["a", g] = exchange(
                sua_ref.at[g], r2a_ref.at[g], 1, 0, g, yn
            )
            p2["a", g].start()
            p1["b", g].wait()
            sub_ref[g] = (
                zkb_ref[g, pl.ds(o_sb, mq), :].astype(f32)
                + r1b_ref[g, pl.ds(o_sb, mq), :].astype(f32)
            ).astype(bf16)
            p2["b", g] = exchange(
                sub_ref.at[g], r2b_ref.at[g], 1, 1, g, xn
            )
            p2["b", g].start()

        ka, kb = {}, {}
        for g in range(G):
            ka[g] = (
                zka_ref[g, pl.ds(o_ka, mq), :].astype(f32)
                + r1a_ref[g, pl.ds(o_ka, mq), :].astype(f32)
            )
            kb[g] = (
                zkb_ref[g, pl.ds(o_kb, mq), :].astype(f32)
                + r1b_ref[g, pl.ds(o_kb, mq), :].astype(f32)
            )

        p3, p41 = {}, {}
        for g in range(G):
            p2["a", g].wait()
            wa = _gelu(ka[g] + r2a_ref[g].astype(f32))
            ga_ref[g] = wa.astype(bf16)
            p3["a", g] = exchange(ga_ref.at[g], r3a_ref.at[g], 2, 0, g, yn)
            p3["a", g].start()
            p41["a", g] = exchange(
                ga_ref.at[g], r4a_ref.at[g, 0], 3, 0, g, xn
            )
            p41["a", g].start()
            out_ref[pl.ds(rx_me + o_ka, mq), ca[g]] = wa

            p2["b", g].wait()
            wb = _gelu(kb[g] + r2b_ref[g].astype(f32))
            gb_ref[g] = wb.astype(bf16)
            p3["b", g] = exchange(gb_ref.at[g], r3b_ref.at[g], 2, 1, g, xn)
            p3["b", g].start()
            p41["b", g] = exchange(
                gb_ref.at[g], r4b_ref.at[g, 0], 3, 1, g, yn
            )
            p41["b", g].start()
            out_ref[pl.ds(ry_me + o_kb, mq), cb[g]] = wb

        p42 = {}
        for g in range(G):
            p3["a", g].wait()
            p42["a", g] = exchange(
                r3a_ref.at[g], r4a_ref.at[g, 1], 4, 0, g, xn
            )
            p42["a", g].start()
            out_ref[pl.ds(rx_me + o_sa, mq), ca[g]] = r3a_ref[g].astype(f32)

            p3["b", g].wait()
            p42["b", g] = exchange(
                r3b_ref.at[g], r4b_ref.at[g, 1], 4, 1, g, yn
            )
            p42["b", g].start()
            out_ref[pl.ds(ry_me + o_sb, mq), cb[g]] = r3b_ref[g].astype(f32)

        for g in range(G):
            p41["a", g].wait()
            out_ref[pl.ds(rx_nb + o_ka, mq), ca[g]] = r4a_ref[g, 0].astype(f32)
            p41["b", g].wait()
            out_ref[pl.ds(ry_nb + o_kb, mq), cb[g]] = r4b_ref[g, 0].astype(f32)
        for g in range(G):
            p42["a", g].wait()
            out_ref[pl.ds(rx_nb + o_sa, mq), ca[g]] = r4a_ref[g, 1].astype(f32)
            p42["b", g].wait()
            out_ref[pl.ds(ry_nb + o_sb, mq), cb[g]] = r4b_ref[g, 1].astype(f32)

    return pl.pallas_call(
        body,
        out_shape=jax.ShapeDtypeStruct((m, n), f32),
        in_specs=[
            pl.BlockSpec(memory_space=pltpu.VMEM),
            pl.BlockSpec(memory_space=pltpu.VMEM),
        ],
        out_specs=pl.BlockSpec(memory_space=pltpu.VMEM),
        scratch_shapes=[
            pltpu.VMEM((G, mh, cw), bf16),
            pltpu.VMEM((G, mh, cw), bf16),
            pltpu.VMEM((G, mh, cw), bf16),
            pltpu.VMEM((G, mh, cw), bf16),
            pltpu.VMEM((G, mh, cw), bf16),
            pltpu.VMEM((G, mh, cw), bf16),
            pltpu.VMEM((G, mq, cw), bf16),
            pltpu.VMEM((G, mq, cw), bf16),
            pltpu.VMEM((G, mq, cw), bf16),
            pltpu.VMEM((G, mq, cw), bf16),
            pltpu.VMEM((G, mq, cw), bf16),
            pltpu.VMEM((G, mq, cw), bf16),
            pltpu.VMEM((G, mq, cw), bf16),
            pltpu.VMEM((G, mq, cw), bf16),
            pltpu.VMEM((G, 2, mq, cw), bf16),
            pltpu.VMEM((G, 2, mq, cw), bf16),
            pltpu.SemaphoreType.DMA((5, 2, G)),
            pltpu.SemaphoreType.DMA((5, 2, G)),
        ],
        compiler_params=pltpu.CompilerParams(collective_id=0),
    )(A, B)


# device time: 28223 ns/iter; 5.3718x vs baseline; 1.0108x over previous
import jax
import jax.numpy as jnp
from jax import lax
from jax.experimental import pallas as pl
from jax.experimental.pallas import tpu as pltpu

N_DEV = 4
G = 4


def _gelu(z):
    return 0.5 * z * (1.0 + jnp.tanh(0.7978845608 * (z + 0.044715 * z * z * z)))


def kernel(A, B):
    m, k = A.shape
    _, n = B.shape
    mh, nh = m // 2, n // 2
    mq = m // 4
    cw = nh // G
    f32, bf16 = jnp.float32, jnp.bfloat16

    def body(
        a_ref, b_ref, out_ref,
        s1a_ref, s1b_ref, zka_ref, zkb_ref, r1a_ref, r1b_ref,
        sua_ref, sub_ref, r2a_ref, r2b_ref,
        ga_ref, gb_ref, r3a_ref, r3b_ref,
        r4a_ref, r4b_ref,
        send_sems, recv_sems,
    ):
        my = lax.axis_index("i")
        mx = my // 2
        myy = (my ^ (my >> 1)) & 1
        xn = my ^ 3
        yn = my ^ 1

        rx_me = mh * mx
        rx_nb = mh * (1 - mx)
        ry_me = mh * myy
        ry_nb = mh * (1 - myy)
        o_sa = mq * (1 - myy)
        o_ka = mq * myy
        o_sb = mq * (1 - mx)
        o_kb = mq * mx
        ca = [slice(g * cw, (g + 1) * cw) for g in range(G)]
        cb = [slice(nh + g * cw, nh + (g + 1) * cw) for g in range(G)]

        barrier = pltpu.get_barrier_semaphore()
        for nbr in (xn, yn):
            pl.semaphore_signal(
                barrier, inc=1,
                device_id=(nbr,), device_id_type=pl.DeviceIdType.MESH,
            )
        pl.semaphore_wait(barrier, 2)

        def exchange(src, dst, ph, path, g, peer):
            return pltpu.make_async_remote_copy(
                src_ref=src, dst_ref=dst,
                send_sem=send_sems.at[ph, path, g],
                recv_sem=recv_sems.at[ph, path, g],
                device_id=(peer,), device_id_type=pl.DeviceIdType.MESH,
            )

        a_xnb = a_ref[pl.ds(rx_nb, mh), :].astype(bf16)
        a_ynb = a_ref[pl.ds(ry_nb, mh), :].astype(bf16)
        a_xme = a_ref[pl.ds(rx_me, mh), :].astype(bf16)
        a_yme = a_ref[pl.ds(ry_me, mh), :].astype(bf16)
        b_bf = b_ref[...].astype(bf16)

        p1 = {}
        for g in range(G):
            s1a_ref[g] = jnp.dot(
                a_xnb, b_bf[:, ca[g]], preferred_element_type=f32
            ).astype(bf16)
            p1["a", g] = exchange(
                s1a_ref.at[g], r1a_ref.at[g], 0, 0, g, xn
            )
            p1["a", g].start()
            s1b_ref[g] = jnp.dot(
                a_ynb, b_bf[:, cb[g]], preferred_element_type=f32
            ).astype(bf16)
            p1["b", g] = exchange(
                s1b_ref.at[g], r1b_ref.at[g], 0, 1, g, yn
            )
            p1["b", g].start()
        for g in range(G):
            zka_ref[g] = jnp.dot(
                a_xme, b_bf[:, ca[g]], preferred_element_type=f32
            ).astype(bf16)
            zkb_ref[g] = jnp.dot(
                a_yme, b_bf[:, cb[g]], preferred_element_type=f32
            ).astype(bf16)

        p2 = {}
        for g in range(G):
            p1["a", g].wait()
            sua_ref[g] = (
                zka_ref[g, pl.ds(o_sa, mq), :].astype(f32)
                + r1a_ref[g, pl.ds(o_sa, mq), :].astype(f32)
            ).astype(bf16)
            p2["a", g] = exchange(
                sua_ref.at[g], r2a_ref.at[g], 1, 0, g, yn
            )
            p2["a", g].start()
            p1["b", g].wait()
            sub_ref[g] = (
                zkb_ref[g, pl.ds(o_sb, mq), :].astype(f32)
                + r1b_ref[g, pl.ds(o_sb, mq), :].astype(f32)
            ).astype(bf16)
            p2["b", g] = exchange(
                sub_ref.at[g], r2b_ref.at[g], 1, 1, g, xn
            )
            p2["b", g].start()

        ka, kb = {}, {}
        for g in range(G):
            ka[g] = (
                zka_ref[g, pl.ds(o_ka, mq), :].astype(f32)
                + r1a_ref[g, pl.ds(o_ka, mq), :].astype(f32)
            )
            kb[g] = (
                zkb_ref[g, pl.ds(o_kb, mq), :].astype(f32)
                + r1b_ref[g, pl.ds(o_kb, mq), :].astype(f32)
            )

        p3, p41 = {}, {}
        for g in range(G):
            p2["a", g].wait()
            wa = _gelu(ka[g] + r2a_ref[g].astype(f32))
            ga_ref[g] = wa.astype(bf16)
            p3["a", g] = exchange(ga_ref.at[g], r3a_ref.at[g], 2, 0, g, yn)
            p3["a", g].start()
            p41["a", g] = exchange(
                ga_ref.at[g], r4a_ref.at[g, 0], 3, 0, g, xn
            )
            p41["a", g].start()
            out_ref[pl.ds(rx_me + o_ka, mq), ca[g]] = wa

            p2["b", g].wait()
            wb = _gelu(kb[g] + r2b_ref[g].astype(f32))
            gb_ref[g] = wb.astype(bf16)
            p3["b", g] = exchange(gb_ref.at[g], r3b_ref.at[g], 2, 1, g, xn)
            p3["b", g].start()
            p41["b", g] = exchange(
                gb_ref.at[g], r4b_ref.at[g, 0], 3, 1, g, yn
            )
            p41["b", g].start()
            out_ref[pl.ds(ry_me + o_kb, mq), cb[g]] = wb

        p42 = {}
        for g in range(G):
            p3["a", g].wait()
            p42["a", g] = exchange(
                r3a_ref.at[g], r4a_ref.at[g, 1], 4, 0, g, xn
            )
            p42["a", g].start()
            out_ref[pl.ds(rx_me + o_sa, mq), ca[g]] = r3a_ref[g].astype(f32)

            p3["b", g].wait()
            p42["b", g] = exchange(
                r3b_ref.at[g], r4b_ref.at[g, 1], 4, 1, g, yn
            )
            p42["b", g].start()
            out_ref[pl.ds(ry_me + o_sb, mq), cb[g]] = r3b_ref[g].astype(f32)

        for g in range(G):
            p41["a", g].wait()
            out_ref[pl.ds(rx_nb + o_ka, mq), ca[g]] = r4a_ref[g, 0].astype(f32)
            p41["b", g].wait()
            out_ref[pl.ds(ry_nb + o_kb, mq), cb[g]] = r4b_ref[g, 0].astype(f32)
        for g in range(G):
            p42["a", g].wait()
            out_ref[pl.ds(rx_nb + o_sa, mq), ca[g]] = r4a_ref[g, 1].astype(f32)
            p42["b", g].wait()
            out_ref[pl.ds(ry_nb + o_sb, mq), cb[g]] = r4b_ref[g, 1].astype(f32)

    return pl.pallas_call(
        body,
        out_shape=jax.ShapeDtypeStruct((m, n), f32),
        in_specs=[
            pl.BlockSpec(memory_space=pltpu.VMEM),
            pl.BlockSpec(memory_space=pltpu.VMEM),
        ],
        out_specs=pl.BlockSpec(memory_space=pltpu.VMEM),
        scratch_shapes=[
            pltpu.VMEM((G, mh, cw), bf16),
            pltpu.VMEM((G, mh, cw), bf16),
            pltpu.VMEM((G, mh, cw), bf16),
            pltpu.VMEM((G, mh, cw), bf16),
            pltpu.VMEM((G, mh, cw), bf16),
            pltpu.VMEM((G, mh, cw), bf16),
            pltpu.VMEM((G, mq, cw), bf16),
            pltpu.VMEM((G, mq, cw), bf16),
            pltpu.VMEM((G, mq, cw), bf16),
            pltpu.VMEM((G, mq, cw), bf16),
            pltpu.VMEM((G, mq, cw), bf16),
            pltpu.VMEM((G, mq, cw), bf16),
            pltpu.VMEM((G, mq, cw), bf16),
            pltpu.VMEM((G, mq, cw), bf16),
            pltpu.VMEM((G, 2, mq, cw), bf16),
            pltpu.VMEM((G, 2, mq, cw), bf16),
            pltpu.SemaphoreType.DMA((5, 2, G)),
            pltpu.SemaphoreType.DMA((5, 2, G)),
        ],
        compiler_params=pltpu.CompilerParams(collective_id=0),
    )(A, B)
